# baseline (device time: 507365 ns/iter reference)
import jax
import jax.numpy as jnp
from jax import lax
from jax.experimental import pallas as pl
from jax.experimental.pallas import tpu as pltpu

T = 2048
D = 4096
VH = 8192
SC = 256
NSC = T // SC
VT = 512
NVT = VH // VT
VT0 = 256
NVT0 = VH // VT0
C = 64
NSUB = SC // C
RSLOTS = 2


def _fused(x, w):

    def body(x_ref, w_ref, out_ref, wbf_ref, xf_ref, xs_ref, wf_ref,
             wc_ref, wt_ref, lg_ref, rv_ref, stg_ref, x_sem, wf_sems,
             wb_sems, wt_sems, send_sems, recv_sems, out_sems,
             credit_sem):
        mx = lax.axis_index("x")
        my = lax.axis_index("y")
        mz = lax.axis_index("z")
        partner = (mx, my, 1 - mz)

        barrier = pltpu.get_barrier_semaphore()
        pl.semaphore_signal(barrier, inc=1, device_id=partner,
                            device_id_type=pl.DeviceIdType.MESH)
        pl.semaphore_wait(barrier, 1)

        def load_x(s):
            return pltpu.make_async_copy(
                x_ref.at[pl.ds(s * SC, SC), :], xf_ref, x_sem)

        def send_chunk(s):
            return pltpu.make_async_remote_copy(
                src_ref=lg_ref.at[s % 2],
                dst_ref=rv_ref.at[s % RSLOTS],
                send_sem=send_sems.at[s % 2],
                recv_sem=recv_sems.at[s % RSLOTS],
                device_id=partner,
                device_id_type=pl.DeviceIdType.MESH,
            )

        def out_dma(k):
            c, sub = divmod(k, NSUB)
            rows = pl.ds(c * SC + sub * C, C)
            return pltpu.make_async_copy(
                stg_ref.at[k % 2], out_ref.at[rows, :], out_sems.at[k % 2])

        def wf_dma(vt, slot):
            return pltpu.make_async_copy(
                w_ref.at[:, pl.ds(vt * VT0, VT0)], wf_ref.at[slot],
                wf_sems.at[slot])

        def wb_dma(vt, slot):
            return pltpu.make_async_copy(
                wc_ref.at[slot], wbf_ref.at[:, pl.ds(vt * VT0, VT0)],
                wb_sems.at[slot])

        def wt_dma(vt, slot):
            return pltpu.make_async_copy(
                wbf_ref.at[:, pl.ds(vt * VT, VT)], wt_ref.at[slot],
                wt_sems.at[slot])

        load_x(0).start()
        wf_dma(0, 0).start()

        for s in range(NSC + 1):
            if s == 0:
                load_x(0).wait()
                xs_ref[...] = xf_ref[...].astype(jnp.bfloat16)
                load_x(1).start()

                def cast_tile(vt, carry):
                    slot = lax.rem(vt, 2)
                    wf_dma(vt, slot).wait()

                    @pl.when(vt < NVT0 - 1)
                    def _():
                        wf_dma(vt + 1, lax.rem(vt + 1, 2)).start()

                    @pl.when(vt >= 2)
                    def _():
                        wb_dma(vt - 2, slot).wait()

                    wc_ref[slot] = wf_ref[slot].astype(jnp.bfloat16)
                    wb_dma(vt, slot).start()
                    acc = jnp.dot(xs_ref[...], wc_ref[slot],
                                  preferred_element_type=jnp.float32)
                    lg_ref[0, :, pl.ds(vt * VT0, VT0)] = (
                        acc.astype(jnp.bfloat16))
                    return carry

                lax.fori_loop(0, NVT0, cast_tile, 0)
                wb_dma(NVT0 - 2, 0).wait()
                wb_dma(NVT0 - 1, 1).wait()

                send_chunk(0).start()

            elif s < NSC:
                if s >= 2:
                    send_chunk(s - 2).wait_send()
                load_x(s).wait()
                xs_ref[...] = xf_ref[...].astype(jnp.bfloat16)
                if s + 1 < NSC:
                    load_x(s + 1).start()
                wt_dma(0, 0).start()
                wt_dma(1, 1).start()

                def mm_tile(vt, carry, lslot=s % 2):
                    wslot = lax.rem(vt, 3)
                    wt_dma(vt, wslot).wait()

                    @pl.when(vt < NVT - 2)
                    def _():
                        wt_dma(vt + 2, lax.rem(vt + 2, 3)).start()

                    acc = jnp.dot(xs_ref[...], wt_ref[wslot],
                                  preferred_element_type=jnp.float32)
                    lg_ref[lslot, :, pl.ds(vt * VT, VT)] = (
                        acc.astype(jnp.bfloat16))
                    return carry

                lax.fori_loop(0, NVT, mm_tile, 0)

                if s >= RSLOTS:
                    pl.semaphore_wait(credit_sem, 1)
                send_chunk(s).start()

            if s >= 1:
                c = s - 1
                send_chunk(c).wait_recv()
                for sub in range(NSUB):
                    k = c * NSUB + sub
                    r0 = sub * C
                    mine = lg_ref[c % 2, r0:r0 + C, :].astype(jnp.float32)
                    theirs = rv_ref[c % RSLOTS, r0:r0 + C, :].astype(
                        jnp.float32)
                    e_m = jnp.exp(mine)
                    e_t = jnp.exp(theirs)
                    r = 1.0 / (jnp.sum(e_m, -1, keepdims=True)
                               + jnp.sum(e_t, -1, keepdims=True))
                    sm_m = (e_m * r).astype(jnp.bfloat16)
                    sm_t = (e_t * r).astype(jnp.bfloat16)

                    if k >= 2:
                        out_dma(k - 2).wait()

                    sslot = k % 2

                    @pl.when(mz == 0)
                    def _(sslot=sslot, sm_m=sm_m, sm_t=sm_t):
                        stg_ref[sslot, :, :VH] = sm_m
                        stg_ref[sslot, :, VH:] = sm_t

                    @pl.when(mz == 1)
                    def _(sslot=sslot, sm_m=sm_m, sm_t=sm_t):
                        stg_ref[sslot, :, :VH] = sm_t
                        stg_ref[sslot, :, VH:] = sm_m

                    out_dma(k).start()

                if c < NSC - RSLOTS:
                    pl.semaphore_signal(credit_sem, inc=1,
                                        device_id=partner,
                                        device_id_type=pl.DeviceIdType.MESH)

        send_chunk(NSC - 2).wait_send()
        send_chunk(NSC - 1).wait_send()
        out_dma(NSC * NSUB - 2).wait()
        out_dma(NSC * NSUB - 1).wait()

    return pl.pallas_call(
        body,
        out_shape=(
            jax.ShapeDtypeStruct((T, 2 * VH), jnp.bfloat16),
            jax.ShapeDtypeStruct((D, VH), jnp.bfloat16),
        ),
        in_specs=[pl.BlockSpec(memory_space=pl.ANY),
                  pl.BlockSpec(memory_space=pl.ANY)],
        out_specs=(pl.BlockSpec(memory_space=pl.ANY),
                   pl.BlockSpec(memory_space=pl.ANY)),
        scratch_shapes=[
            pltpu.VMEM((SC, D), jnp.float32),
            pltpu.VMEM((SC, D), jnp.bfloat16),
            pltpu.VMEM((2, D, VT0), jnp.float32),
            pltpu.VMEM((2, D, VT0), jnp.bfloat16),
            pltpu.VMEM((3, D, VT), jnp.bfloat16),
            pltpu.VMEM((2, SC, VH), jnp.bfloat16),
            pltpu.VMEM((RSLOTS, SC, VH), jnp.bfloat16),
            pltpu.VMEM((2, C, 2 * VH), jnp.bfloat16),
            pltpu.SemaphoreType.DMA,
            pltpu.SemaphoreType.DMA((2,)),
            pltpu.SemaphoreType.DMA((2,)),
            pltpu.SemaphoreType.DMA((3,)),
            pltpu.SemaphoreType.DMA((2,)),
            pltpu.SemaphoreType.DMA((RSLOTS,)),
            pltpu.SemaphoreType.DMA((2,)),
            pltpu.SemaphoreType.REGULAR,
        ],
        compiler_params=pltpu.CompilerParams(
            collective_id=0,
            vmem_limit_bytes=60 * 1024 * 1024,
        ),
    )(x, w)


def kernel(x, W):
    out, _ = _fused(x, W)
    return out


# device time: 483651 ns/iter; 1.0490x vs baseline; 1.0490x over previous
import jax
import jax.numpy as jnp
from jax import lax
from jax.experimental import pallas as pl
from jax.experimental.pallas import tpu as pltpu

T = 2048
D = 4096
VH = 8192
SC = 256
NSC = T // SC
VT = 512
NVT = VH // VT
RT = 256
NRT = D // RT
C = 32
NSUB = SC // C
RSLOTS = 2


def _fused(x, w):

    def body(x_ref, w_ref, out_ref, wbf_ref, xf_ref, xs_ref, rf_ref,
             rb_ref, wt_ref, lg_ref, rv_ref, stg_ref, x_sem, rf_sems,
             rb_sem, wt_sems, send_sems, recv_sems, out_sems,
             credit_sem):
        mx = lax.axis_index("x")
        my = lax.axis_index("y")
        mz = lax.axis_index("z")
        partner = (mx, my, 1 - mz)

        barrier = pltpu.get_barrier_semaphore()
        pl.semaphore_signal(barrier, inc=1, device_id=partner,
                            device_id_type=pl.DeviceIdType.MESH)
        pl.semaphore_wait(barrier, 1)

        def load_x(s):
            return pltpu.make_async_copy(
                x_ref.at[pl.ds(s * SC, SC), :], xf_ref, x_sem)

        def rf_dma(rt, slot):
            return pltpu.make_async_copy(
                w_ref.at[pl.ds(rt * RT, RT), :], rf_ref.at[slot],
                rf_sems.at[slot])

        def rb_dma(rt):
            return pltpu.make_async_copy(
                rb_ref, wbf_ref.at[pl.ds(rt * RT, RT), :], rb_sem)

        def wt_dma(vt, slot):
            return pltpu.make_async_copy(
                wbf_ref.at[:, pl.ds(vt * VT, VT)], wt_ref.at[slot],
                wt_sems.at[slot])

        def send_chunk(s):
            return pltpu.make_async_remote_copy(
                src_ref=lg_ref.at[s % 2],
                dst_ref=rv_ref.at[s % RSLOTS],
                send_sem=send_sems.at[s % 2],
                recv_sem=recv_sems.at[s % RSLOTS],
                device_id=partner,
                device_id_type=pl.DeviceIdType.MESH,
            )

        def out_dma(k):
            c, sub = divmod(k, NSUB)
            rows = pl.ds(c * SC + sub * C, C)
            return pltpu.make_async_copy(
                stg_ref.at[k % 2], out_ref.at[rows, :], out_sems.at[k % 2])

        load_x(0).start()
        rf_dma(0, 0).start()
        rf_dma(1, 1).start()

        def cast_tile(rt, carry):
            slot = lax.rem(rt, 2)
            rf_dma(rt, slot).wait()

            @pl.when(rt >= 1)
            def _():
                rb_dma(rt - 1).wait()

            rb_ref[...] = rf_ref[slot].astype(jnp.bfloat16)
            rb_dma(rt).start()

            @pl.when(rt < NRT - 2)
            def _():
                rf_dma(rt + 2, slot).start()

            return carry

        lax.fori_loop(0, NRT, cast_tile, 0)
        rb_dma(NRT - 1).wait()

        for s in range(NSC + 1):
            if s < NSC:
                if s >= 2:
                    send_chunk(s - 2).wait_send()
                load_x(s).wait()
                xs_ref[...] = xf_ref[...].astype(jnp.bfloat16)
                if s + 1 < NSC:
                    load_x(s + 1).start()
                wt_dma(0, 0).start()
                wt_dma(1, 1).start()

                send_partial = s == 0

                def mm_tile(vt, carry, lslot=s % 2,
                            send_partial=send_partial):
                    wslot = lax.rem(vt, 3)
                    wt_dma(vt, wslot).wait()

                    @pl.when(vt < NVT - 2)
                    def _():
                        wt_dma(vt + 2, lax.rem(vt + 2, 3)).start()

                    acc = jnp.dot(xs_ref[...], wt_ref[wslot],
                                  preferred_element_type=jnp.float32)
                    lg_ref[lslot, :, pl.ds(vt * VT, VT)] = (
                        acc.astype(jnp.bfloat16))
                    if send_partial:
                        cols = pl.ds(vt * VT, VT)
                        pltpu.make_async_remote_copy(
                            src_ref=lg_ref.at[0, :, cols],
                            dst_ref=rv_ref.at[0, :, cols],
                            send_sem=send_sems.at[0],
                            recv_sem=recv_sems.at[0],
                            device_id=partner,
                            device_id_type=pl.DeviceIdType.MESH,
                        ).start()
                    return carry

                lax.fori_loop(0, NVT, mm_tile, 0)

                if not send_partial:
                    if s >= RSLOTS:
                        pl.semaphore_wait(credit_sem, 1)
                    send_chunk(s).start()

            if s >= 1:
                c = s - 1
                send_chunk(c).wait_recv()
                for sub in range(NSUB):
                    k = c * NSUB + sub
                    r0 = sub * C
                    mine = lg_ref[c % 2, r0:r0 + C, :].astype(jnp.float32)
                    theirs = rv_ref[c % RSLOTS, r0:r0 + C, :].astype(
                        jnp.float32)
                    e_m = jnp.exp(mine)
                    e_t = jnp.exp(theirs)
                    r = 1.0 / (jnp.sum(e_m, -1, keepdims=True)
                               + jnp.sum(e_t, -1, keepdims=True))
                    sm_m = (e_m * r).astype(jnp.bfloat16)
                    sm_t = (e_t * r).astype(jnp.bfloat16)

                    if k >= 2:
                        out_dma(k - 2).wait()

                    sslot = k % 2

                    @pl.when(mz == 0)
                    def _(sslot=sslot, sm_m=sm_m, sm_t=sm_t):
                        stg_ref[sslot, :, :VH] = sm_m
                        stg_ref[sslot, :, VH:] = sm_t

                    @pl.when(mz == 1)
                    def _(sslot=sslot, sm_m=sm_m, sm_t=sm_t):
                        stg_ref[sslot, :, :VH] = sm_t
                        stg_ref[sslot, :, VH:] = sm_m

                    out_dma(k).start()

                if c < NSC - RSLOTS:
                    pl.semaphore_signal(credit_sem, inc=1,
                                        device_id=partner,
                                        device_id_type=pl.DeviceIdType.MESH)

        send_chunk(NSC - 2).wait_send()
        send_chunk(NSC - 1).wait_send()
        out_dma(NSC * NSUB - 2).wait()
        out_dma(NSC * NSUB - 1).wait()

    return pl.pallas_call(
        body,
        out_shape=(
            jax.ShapeDtypeStruct((T, 2 * VH), jnp.bfloat16),
            jax.ShapeDtypeStruct((D, VH), jnp.bfloat16),
        ),
        in_specs=[pl.BlockSpec(memory_space=pl.ANY),
                  pl.BlockSpec(memory_space=pl.ANY)],
        out_specs=(pl.BlockSpec(memory_space=pl.ANY),
                   pl.BlockSpec(memory_space=pl.ANY)),
        scratch_shapes=[
            pltpu.VMEM((SC, D), jnp.float32),
            pltpu.VMEM((SC, D), jnp.bfloat16),
            pltpu.VMEM((2, RT, VH), jnp.float32),
            pltpu.VMEM((RT, VH), jnp.bfloat16),
            pltpu.VMEM((3, D, VT), jnp.bfloat16),
            pltpu.VMEM((2, SC, VH), jnp.bfloat16),
            pltpu.VMEM((RSLOTS, SC, VH), jnp.bfloat16),
            pltpu.VMEM((2, C, 2 * VH), jnp.bfloat16),
            pltpu.SemaphoreType.DMA,
            pltpu.SemaphoreType.DMA((2,)),
            pltpu.SemaphoreType.DMA,
            pltpu.SemaphoreType.DMA((3,)),
            pltpu.SemaphoreType.DMA((2,)),
            pltpu.SemaphoreType.DMA((RSLOTS,)),
            pltpu.SemaphoreType.DMA((2,)),
            pltpu.SemaphoreType.REGULAR,
        ],
        compiler_params=pltpu.CompilerParams(
            collective_id=0,
            vmem_limit_bytes=60 * 1024 * 1024,
        ),
    )(x, w)


def kernel(x, W):
    out, _ = _fused(x, W)
    return out


# device time: 454557 ns/iter; 1.1162x vs baseline; 1.0640x over previous
import jax
import jax.numpy as jnp
from jax import lax
from jax.experimental import pallas as pl
from jax.experimental.pallas import tpu as pltpu

T = 2048
D = 4096
VH = 8192
SC = 256
NSC = T // SC
VT = 512
NVT = VH // VT
C = 32
NSUB = SC // C
RSLOTS = 2


def _fused(x, w):

    def body(x_ref, w_ref, out_ref, wbf_ref, xf_ref, xs_ref, wf_ref,
             wt_ref, lg_ref, rv_ref, stg_ref, x_sem, wf_sems, wb_sems,
             wt_sems, send_sems, recv_sems, out_sems, credit_sem):
        mx = lax.axis_index("x")
        my = lax.axis_index("y")
        mz = lax.axis_index("z")
        partner = (mx, my, 1 - mz)

        barrier = pltpu.get_barrier_semaphore()
        pl.semaphore_signal(barrier, inc=1, device_id=partner,
                            device_id_type=pl.DeviceIdType.MESH)
        pl.semaphore_wait(barrier, 1)

        def load_x(s):
            return pltpu.make_async_copy(
                x_ref.at[pl.ds(s * SC, SC), :], xf_ref, x_sem)

        def wf_dma(vt, slot):
            return pltpu.make_async_copy(
                w_ref.at[:, pl.ds(vt * VT, VT)], wf_ref.at[slot],
                wf_sems.at[slot])

        def wb_dma(vt):
            slot = lax.rem(vt, 3)
            return pltpu.make_async_copy(
                wt_ref.at[slot], wbf_ref.at[vt], wb_sems.at[slot])

        def wt_dma(vt, slot):
            return pltpu.make_async_copy(
                wbf_ref.at[vt], wt_ref.at[slot], wt_sems.at[slot])

        def send_chunk(s):
            return pltpu.make_async_remote_copy(
                src_ref=lg_ref.at[s % 2],
                dst_ref=rv_ref.at[s % RSLOTS],
                send_sem=send_sems.at[s % 2],
                recv_sem=recv_sems.at[s % RSLOTS],
                device_id=partner,
                device_id_type=pl.DeviceIdType.MESH,
            )

        def send_piece(c, cols):
            pltpu.make_async_remote_copy(
                src_ref=lg_ref.at[c, :, cols],
                dst_ref=rv_ref.at[c, :, cols],
                send_sem=send_sems.at[c],
                recv_sem=recv_sems.at[c],
                device_id=partner,
                device_id_type=pl.DeviceIdType.MESH,
            ).start()

        def out_dma(k):
            c, sub = divmod(k, NSUB)
            rows = pl.ds(c * SC + sub * C, C)
            return pltpu.make_async_copy(
                stg_ref.at[k % 2], out_ref.at[rows, :], out_sems.at[k % 2])

        def consume(c):
            send_chunk(c).wait_recv()
            for sub in range(NSUB):
                k = c * NSUB + sub
                r0 = sub * C
                mine = lg_ref[c % 2, r0:r0 + C, :].astype(jnp.float32)
                theirs = rv_ref[c % RSLOTS, r0:r0 + C, :].astype(
                    jnp.float32)
                e_m = jnp.exp(mine)
                e_t = jnp.exp(theirs)
                r = 1.0 / (jnp.sum(e_m, -1, keepdims=True)
                           + jnp.sum(e_t, -1, keepdims=True))
                sm_m = (e_m * r).astype(jnp.bfloat16)
                sm_t = (e_t * r).astype(jnp.bfloat16)

                if k >= 2:
                    out_dma(k - 2).wait()

                sslot = k % 2

                @pl.when(mz == 0)
                def _(sslot=sslot, sm_m=sm_m, sm_t=sm_t):
                    stg_ref[sslot, :, :VH] = sm_m
                    stg_ref[sslot, :, VH:] = sm_t

                @pl.when(mz == 1)
                def _(sslot=sslot, sm_m=sm_m, sm_t=sm_t):
                    stg_ref[sslot, :, :VH] = sm_t
                    stg_ref[sslot, :, VH:] = sm_m

                out_dma(k).start()

            if c < NSC - RSLOTS:
                pl.semaphore_signal(credit_sem, inc=1,
                                    device_id=partner,
                                    device_id_type=pl.DeviceIdType.MESH)

        load_x(0).start()
        wf_dma(0, 0).start()
        wf_dma(1, 1).start()
        load_x(0).wait()
        xs_ref[0] = xf_ref[...].astype(jnp.bfloat16)
        load_x(1).start()
        load_x(1).wait()
        xs_ref[1] = xf_ref[...].astype(jnp.bfloat16)
        load_x(2).start()

        def cast_mm_tile(vt, carry):
            fslot = lax.rem(vt, 2)
            wslot = lax.rem(vt, 3)
            cols = pl.ds(vt * VT, VT)
            wf_dma(vt, fslot).wait()

            @pl.when(vt >= 3)
            def _():
                wb_dma(vt - 3).wait()

            wt_ref[wslot] = wf_ref[fslot].astype(jnp.bfloat16)

            @pl.when(vt < NVT - 2)
            def _():
                wf_dma(vt + 2, fslot).start()

            wb_dma(vt).start()
            for c in range(2):
                acc = jnp.dot(xs_ref[c], wt_ref[wslot],
                              preferred_element_type=jnp.float32)
                lg_ref[c, :, cols] = acc.astype(jnp.bfloat16)
                send_piece(c, cols)
            return carry

        lax.fori_loop(0, NVT, cast_mm_tile, 0)
        wb_dma(NVT - 3).wait()
        wb_dma(NVT - 2).wait()
        wb_dma(NVT - 1).wait()

        consume(0)

        for s in range(2, NSC):
            send_chunk(s - 2).wait_send()
            load_x(s).wait()
            xs_ref[s % 2] = xf_ref[...].astype(jnp.bfloat16)
            if s + 1 < NSC:
                load_x(s + 1).start()
            wt_dma(0, 0).start()
            wt_dma(1, 1).start()

            def mm_tile(vt, carry, lslot=s % 2):
                wslot = lax.rem(vt, 3)
                wt_dma(vt, wslot).wait()

                @pl.when(vt < NVT - 2)
                def _():
                    wt_dma(vt + 2, lax.rem(vt + 2, 3)).start()

                acc = jnp.dot(xs_ref[lslot], wt_ref[wslot],
                              preferred_element_type=jnp.float32)
                lg_ref[lslot, :, pl.ds(vt * VT, VT)] = (
                    acc.astype(jnp.bfloat16))
                return carry

            lax.fori_loop(0, NVT, mm_tile, 0)

            pl.semaphore_wait(credit_sem, 1)
            send_chunk(s).start()
            consume(s - 1)

        consume(NSC - 1)

        send_chunk(NSC - 2).wait_send()
        send_chunk(NSC - 1).wait_send()
        out_dma(NSC * NSUB - 2).wait()
        out_dma(NSC * NSUB - 1).wait()

    return pl.pallas_call(
        body,
        out_shape=(
            jax.ShapeDtypeStruct((T, 2 * VH), jnp.bfloat16),
            jax.ShapeDtypeStruct((NVT, D, VT), jnp.bfloat16),
        ),
        in_specs=[pl.BlockSpec(memory_space=pl.ANY),
                  pl.BlockSpec(memory_space=pl.ANY)],
        out_specs=(pl.BlockSpec(memory_space=pl.ANY),
                   pl.BlockSpec(memory_space=pl.ANY)),
        scratch_shapes=[
            pltpu.VMEM((SC, D), jnp.float32),
            pltpu.VMEM((2, SC, D), jnp.bfloat16),
            pltpu.VMEM((2, D, VT), jnp.float32),
            pltpu.VMEM((3, D, VT), jnp.bfloat16),
            pltpu.VMEM((2, SC, VH), jnp.bfloat16),
            pltpu.VMEM((RSLOTS, SC, VH), jnp.bfloat16),
            pltpu.VMEM((2, C, 2 * VH), jnp.bfloat16),
            pltpu.SemaphoreType.DMA,
            pltpu.SemaphoreType.DMA((2,)),
            pltpu.SemaphoreType.DMA((3,)),
            pltpu.SemaphoreType.DMA((3,)),
            pltpu.SemaphoreType.DMA((2,)),
            pltpu.SemaphoreType.DMA((RSLOTS,)),
            pltpu.SemaphoreType.DMA((2,)),
            pltpu.SemaphoreType.REGULAR,
        ],
        compiler_params=pltpu.CompilerParams(
            collective_id=0,
            vmem_limit_bytes=60 * 1024 * 1024,
        ),
    )(x, w)


def kernel(x, W):
    out, _ = _fused(x, W)
    return out
